# baseline (speedup 1.0000x reference)
import functools

import jax
import jax.numpy as jnp
from jax.experimental import pallas as pl
from jax.experimental.pallas import tpu as pltpu

NREAL = 307
NP = 320
T = 12
FC = 64
FT = 64
KCH = 3
P = 12

_INTERPRET = False

_BKEYS = ['u1', 'u2', 'u3', 'be', 've', 'w1', 'w2', 'w3', 'bs', 'vs',
          'cw', 'cb', 'tw', 'tb', 'rw', 'rb', 'lng', 'lnb']

_dot = functools.partial(jnp.dot, preferred_element_type=jnp.float32,
                         precision=jax.lax.Precision.HIGHEST)


def _dg_t(a, b):
    return jax.lax.dot_general(a, b, (((1,), (1,)), ((), ())),
                               preferred_element_type=jnp.float32,
                               precision=jax.lax.Precision.HIGHEST)


def _softmax0(m):
    mx = jnp.max(m, axis=0, keepdims=True)
    e = jnp.exp(m - mx)
    return e / jnp.sum(e, axis=0, keepdims=True)


def _graph_kernel(row_ref, col_ref, a_ref):
    row = row_ref[...]
    col = col_ref[...]
    maskf = (row != col).astype(jnp.float32)
    ep = row.shape[1]
    ion = jax.lax.broadcasted_iota(jnp.int32, (NP, ep), 0)
    rt = (row == ion).astype(jnp.float32)
    ct = (col == ion).astype(jnp.float32)
    deg = jnp.sum(rt * maskf, axis=1, keepdims=True)
    dinv = jnp.where(deg > 0, jax.lax.rsqrt(jnp.maximum(deg, 1.0)), 0.0)
    dr = jnp.sum(rt * dinv, axis=0, keepdims=True)
    dc = jnp.sum(ct * dinv, axis=0, keepdims=True)
    nv = -dr * dc * maskf
    a_ref[...] = _dg_t(ct * nv, rt)


def _block(Xc, F, r, A):
    u1X = _dot(r['u1'][...], Xc)
    LHS1 = jnp.concatenate([u1X[:, t * F:(t + 1) * F] for t in range(T)],
                           axis=0)
    LHS = _dot(LHS1, r['u2'][...])
    io_r = jax.lax.broadcasted_iota(jnp.int32, (F * T, T), 0)
    io_c = jax.lax.broadcasted_iota(jnp.int32, (F * T, T), 1)
    seg = io_r // F == io_c
    u3t = jnp.concatenate([r['u3'][...]] * T, axis=0)
    RHSm = _dot(Xc, jnp.where(seg, u3t, 0.0))
    P1 = _dot(LHS, RHSm)
    Emat = _dot(r['ve'][...], jax.nn.sigmoid(P1 + r['be'][...]))
    Et = _softmax0(Emat)

    ew = _dot(Et, r['w1'][...])
    if F == 1:
        XW = _dot(Xc, ew)
    else:
        XW = Xc[:, 0:F] * ew[0:1, 0:1]
        for s in range(1, T):
            XW = XW + Xc[:, s * F:(s + 1) * F] * ew[s:s + 1, 0:1]
    LHSs = _dot(XW, r['w2'][...])
    w3t = jnp.concatenate([r['w3'][...]] * T, axis=0)
    Cw3 = _dot(Xc, jnp.where(seg, w3t, 0.0))
    Rm = _dot(Cw3, Et)
    P2 = _dg_t(LHSs, Rm)
    Q = jax.nn.sigmoid(P2 + r['bs'][...])
    Spre = _dot(r['vs'][...], Q)
    io0 = jax.lax.broadcasted_iota(jnp.int32, (NP, NP), 0)
    io1 = jax.lax.broadcasted_iota(jnp.int32, (NP, NP), 1)
    S = _softmax0(jnp.where(io0 < NREAL, Spre, -1e9))
    d = jnp.sum(jnp.where(io0 == io1, S, 0.0), axis=1, keepdims=True)

    St = S.T
    Tx0 = d * Xc
    Tx1 = _dot(A * St, Tx0)
    Tx2 = 2.0 * _dot(A, Tx1) - Tx0
    cwr = r['cw'][...]
    cb = r['cb'][...]
    sg = []
    for t in range(T):
        sl = slice(t * F, (t + 1) * F)
        cat = jnp.concatenate([Tx0[:, sl], Tx1[:, sl], Tx2[:, sl]], axis=1)
        sg.append(jnp.maximum(_dot(cat, cwr) + cb, 0.0))

    wcat = jnp.concatenate([r['tw'][...], r['rw'][...]], axis=0)
    bias = r['tb'][...] + r['rb'][...]
    Z = jnp.zeros((NP, FC), jnp.float32)
    g = r['lng'][...]
    bb = r['lnb'][...]
    outs = []
    for t in range(T):
        left = sg[t - 1] if t > 0 else Z
        right = sg[t + 1] if t < T - 1 else Z
        cat4 = jnp.concatenate([left, sg[t], right,
                                Xc[:, t * F:(t + 1) * F]], axis=1)
        H = jnp.maximum(_dot(cat4, wcat) + bias, 0.0)
        mu = jnp.mean(H, axis=1, keepdims=True)
        xc = H - mu
        var = jnp.mean(xc * xc, axis=1, keepdims=True)
        outs.append(xc * jax.lax.rsqrt(var + 1e-5) * g + bb)
    return jnp.concatenate(outs, axis=1)


def _main_kernel(*refs):
    x_ref = refs[0]
    a_ref = refs[1]
    b0 = dict(zip(_BKEYS, refs[2:20]))
    b1 = dict(zip(_BKEYS, refs[20:38]))
    fw_ref, fb_ref = refs[38], refs[39]
    out_ref = refs[40]

    A = a_ref[...]
    X0 = x_ref[0]
    X1 = _block(X0, 1, b0, A)
    X2 = _block(X1, FT, b1, A)
    out = jnp.maximum(_dot(X2, fw_ref[...]) + fb_ref[...], 0.0)
    out_ref[0] = out


def _padn(a, axis):
    pad = [(0, 0)] * a.ndim
    pad[axis] = (0, NP - a.shape[axis])
    return jnp.pad(a, pad)


def _prep_block(p, F):
    return {
        'u1': _padn(p['U1'].reshape(1, -1), 1),
        'u2': _padn(p['U2'], 1),
        'u3': p['U3'].reshape(F, 1),
        'be': p['be'][0],
        've': p['Ve'],
        'w1': p['W1'].reshape(T, 1),
        'w2': p['W2'],
        'w3': p['W3'].reshape(F, 1),
        'bs': _padn(_padn(p['bs'][0], 0), 1),
        'vs': _padn(_padn(p['Vs'], 0), 1),
        'cw': p['cw'].reshape(KCH * F, FC),
        'cb': p['cb'].reshape(1, FC),
        'tw': jnp.transpose(p['tw'][:, :, 0, :], (2, 1, 0)).reshape(3 * FC, FT),
        'tb': p['tb'].reshape(1, FT),
        'rw': p['rw'][:, :, 0, 0].T,
        'rb': p['rb'].reshape(1, FT),
        'lng': p['lng'].reshape(1, FT),
        'lnb': p['lnb'].reshape(1, FT),
    }


def _full_spec(shape):
    nz = len(shape)
    return pl.BlockSpec(shape, lambda b, _n=nz: (0,) * _n)


def kernel(x, edge_index, params):
    B = x.shape[0]
    E = edge_index.shape[1]
    EP = ((E + 127) // 128) * 128
    x3 = _padn(x[:, :, 0, :], 1)
    rowp = jnp.pad(edge_index[0], (0, EP - E)).reshape(1, EP)
    colp = jnp.pad(edge_index[1], (0, EP - E)).reshape(1, EP)

    A = pl.pallas_call(
        _graph_kernel,
        out_shape=jax.ShapeDtypeStruct((NP, NP), jnp.float32),
        interpret=_INTERPRET,
    )(rowp, colp)

    b0 = _prep_block(params['b0'], 1)
    b1 = _prep_block(params['b1'], FT)
    fwr = jnp.transpose(params['fw'][:, :, 0, :], (1, 2, 0)).reshape(T * FT, P)
    fbr = params['fb'].reshape(1, P)
    flat = [b0[k] for k in _BKEYS] + [b1[k] for k in _BKEYS] + [fwr, fbr]

    in_specs = [pl.BlockSpec((1, NP, T), lambda b: (b, 0, 0)),
                _full_spec((NP, NP))]
    in_specs += [_full_spec(a.shape) for a in flat]

    outp = pl.pallas_call(
        _main_kernel,
        grid=(B,),
        in_specs=in_specs,
        out_specs=pl.BlockSpec((1, NP, P), lambda b: (b, 0, 0)),
        out_shape=jax.ShapeDtypeStruct((B, NP, P), jnp.float32),
        compiler_params=pltpu.CompilerParams(
            dimension_semantics=("parallel",)),
        interpret=_INTERPRET,
    )(x3, A, *flat)
    return outp[:, :NREAL, :]


# speedup vs baseline: 2.3015x; 2.3015x over previous
import functools

import jax
import jax.numpy as jnp
from jax import lax
from jax.experimental import pallas as pl
from jax.experimental.pallas import tpu as pltpu
from jax.experimental.pallas import tpu_sc as plsc

NREAL = 307
NP = 320
T = 12
FC = 64
FT = 64
KCH = 3
P = 12

_BKEYS = ['u1', 'u2', 'u3', 'be', 've', 'w1', 'w2', 'w3', 'bs', 'vs',
          'cw', 'cb', 'tw', 'tb', 'rw', 'rb', 'lng', 'lnb']
_CKEYS = ['negrow', 'eye', 'sel', 'bsel']

_dot = functools.partial(jnp.dot, preferred_element_type=jnp.float32,
                         precision=jax.lax.Precision.DEFAULT)


def _dg_t(a, b):
    return jax.lax.dot_general(a, b, (((1,), (1,)), ((), ())),
                               preferred_element_type=jnp.float32,
                               precision=jax.lax.Precision.DEFAULT)


def _softmax0(m):
    mx = jnp.max(m, axis=0, keepdims=True)
    e = jnp.exp(m - mx)
    return e / jnp.sum(e, axis=0, keepdims=True)


def _newton_rsqrt(x):
    bits = jax.lax.bitcast_convert_type(x, jnp.int32)
    i = jnp.int32(0x5F3759DF) - jax.lax.shift_right_logical(bits, 1)
    y = jax.lax.bitcast_convert_type(i, jnp.float32)
    for _ in range(3):
        y = y * (1.5 - 0.5 * x * y * y)
    return y


def _graph_sc_body(row_hbm, col_hbm, za_hbm, zd_hbm, a_hbm,
                   row_v, col_v, ridx_v, fidx_v, ones_v, nv_v, dr_v, dc_v,
                   deg_v, dinv_v, deg_sh, dinv_sh, a_sh):
    ep = row_v.shape[0]
    wid = lax.axis_index("s") * 2 + lax.axis_index("c")

    @pl.when(wid == 0)
    def _():
        pltpu.sync_copy(row_hbm, row_v)
        pltpu.sync_copy(col_hbm, col_v)
        pltpu.sync_copy(za_hbm, a_sh)
        pltpu.sync_copy(zd_hbm, deg_sh)
        onec = jnp.ones((16,), jnp.float32)
        for i in range(ep // 16):
            rv = row_v[pl.ds(i * 16, 16)]
            cv = col_v[pl.ds(i * 16, 16)]
            m = rv != cv
            ridx_v[pl.ds(i * 16, 16)] = jnp.where(m, rv, NP)
            fidx_v[pl.ds(i * 16, 16)] = jnp.where(m, cv * NP + rv, NP * NP)
            ones_v[pl.ds(i * 16, 16)] = onec
        pltpu.sync_copy(ones_v, deg_sh.at[ridx_v], add=True)
        pltpu.sync_copy(deg_sh.at[pl.ds(0, NP)], deg_v)
        for j in range(NP // 16):
            dg = deg_v[pl.ds(j * 16, 16)]
            dinv_v[pl.ds(j * 16, 16)] = jnp.where(
                dg > 0, _newton_rsqrt(dg), 0.0)
        pltpu.sync_copy(dinv_v, dinv_sh)
        pltpu.sync_copy(dinv_sh.at[row_v], dr_v)
        pltpu.sync_copy(dinv_sh.at[col_v], dc_v)
        for i in range(ep // 16):
            nv_v[pl.ds(i * 16, 16)] = -(dr_v[pl.ds(i * 16, 16)]
                                        * dc_v[pl.ds(i * 16, 16)])
        pltpu.sync_copy(nv_v, a_sh.at[fidx_v], add=True)
        pltpu.sync_copy(a_sh.at[pl.ds(0, NP * NP)], a_hbm)


def _graph_adjacency_sc(rowp, colp, ep):
    za = jnp.zeros((NP * NP + 16,), jnp.float32)
    zd = jnp.zeros((NP + 16,), jnp.float32)
    mesh = plsc.VectorSubcoreMesh(core_axis_name="c", subcore_axis_name="s")
    fn = functools.partial(
        pl.kernel,
        mesh=mesh,
        out_type=jax.ShapeDtypeStruct((NP * NP,), jnp.float32),
        scratch_types=[
            pltpu.VMEM((ep,), jnp.int32),
            pltpu.VMEM((ep,), jnp.int32),
            pltpu.VMEM((ep,), jnp.int32),
            pltpu.VMEM((ep,), jnp.int32),
            pltpu.VMEM((ep,), jnp.float32),
            pltpu.VMEM((ep,), jnp.float32),
            pltpu.VMEM((ep,), jnp.float32),
            pltpu.VMEM((ep,), jnp.float32),
            pltpu.VMEM((NP,), jnp.float32),
            pltpu.VMEM((NP,), jnp.float32),
            pltpu.VMEM_SHARED((NP + 16,), jnp.float32),
            pltpu.VMEM_SHARED((NP,), jnp.float32),
            pltpu.VMEM_SHARED((NP * NP + 16,), jnp.float32),
        ],
    )(_graph_sc_body)
    return fn(rowp, colp, za, zd).reshape(NP, NP)


def _block(Xc, F, r, A, c):
    u1X = _dot(r['u1'][...], Xc)
    LHS1 = jnp.concatenate([u1X[:, t * F:(t + 1) * F] for t in range(T)],
                           axis=0)
    LHS = _dot(LHS1, r['u2'][...])
    RHSm = _dot(Xc, r['u3'][...])
    P1 = _dot(LHS, RHSm)
    Emat = _dot(r['ve'][...], jax.nn.sigmoid(P1 + r['be'][...]))
    Et = _softmax0(Emat)

    ew = _dot(Et, r['w1'][...])
    if F == 1:
        XW = _dot(Xc, ew)
    else:
        XW = Xc[:, 0:F] * ew[0:1, 0:1]
        for s in range(1, T):
            XW = XW + Xc[:, s * F:(s + 1) * F] * ew[s:s + 1, 0:1]
    LHSs = _dot(XW, r['w2'][...])
    Cw3 = _dot(Xc, r['w3'][...])
    Rm = _dot(Cw3, Et)
    P2T = _dg_t(Rm, LHSs)
    QT = jax.nn.sigmoid(P2T + r['bs'][...])
    SpreT = _dot(QT, r['vs'][...]) + c['negrow'][...]
    mx = jnp.max(SpreT, axis=1, keepdims=True)
    ex = jnp.exp(SpreT - mx)
    St = ex / jnp.sum(ex, axis=1, keepdims=True)
    d = jnp.sum(St * c['eye'][...], axis=1, keepdims=True)

    Tx0 = d * Xc
    Tx1 = _dot(A * St, Tx0)
    Tx2 = 2.0 * _dot(A, Tx1) - Tx0
    cwr = r['cw'][...]
    cb = r['cb'][...]
    cw0, cw1, cw2 = cwr[0:F], cwr[F:2 * F], cwr[2 * F:3 * F]
    sg = []
    for t in range(T):
        sl = slice(t * F, (t + 1) * F)
        acc = _dot(Tx0[:, sl], cw0) + _dot(Tx1[:, sl], cw1) \
            + _dot(Tx2[:, sl], cw2) + cb
        sg.append(jnp.maximum(acc, 0.0))

    twr = r['tw'][...]
    tw0, tw1, tw2 = twr[0:FC], twr[FC:2 * FC], twr[2 * FC:3 * FC]
    rw = r['rw'][...]
    bias = r['tb'][...] + r['rb'][...]
    hs = []
    for t in range(T):
        acc = _dot(sg[t], tw1) + _dot(Xc[:, t * F:(t + 1) * F], rw) + bias
        if t > 0:
            acc = acc + _dot(sg[t - 1], tw0)
        if t < T - 1:
            acc = acc + _dot(sg[t + 1], tw2)
        hs.append(jnp.maximum(acc, 0.0))
    Hcat = jnp.concatenate(hs, axis=1)
    sel = c['sel'][...]
    mus = _dot(Hcat, sel)
    mub = _dot(mus, c['bsel'][...])
    xc = Hcat - mub
    var = _dot(xc * xc, sel)
    rstd = jax.lax.rsqrt(var + 1e-5)
    rsb = _dot(rstd, c['bsel'][...])
    return xc * rsb * r['lng'][...] + r['lnb'][...]


def _main_kernel(*refs):
    x_ref = refs[0]
    a_ref = refs[1]
    b0 = dict(zip(_BKEYS, refs[2:20]))
    b1 = dict(zip(_BKEYS, refs[20:38]))
    fw_ref, fb_ref = refs[38], refs[39]
    c = dict(zip(_CKEYS, refs[40:44]))
    out_ref = refs[44]

    A = a_ref[...]
    for i in range(x_ref.shape[0]):
        X0 = x_ref[i]
        X1 = _block(X0, 1, b0, A, c)
        X2 = _block(X1, FT, b1, A, c)
        out_ref[i] = jnp.maximum(_dot(X2, fw_ref[...]) + fb_ref[...], 0.0)


def _padn(a, axis):
    pad = [(0, 0)] * a.ndim
    pad[axis] = (0, NP - a.shape[axis])
    return jnp.pad(a, pad)


def _blockdiag(v, F):
    import numpy as _np
    idx = _np.arange(F * T) // F
    m = (idx[:, None] == _np.arange(T)[None, :]).astype(_np.float32)
    return jnp.asarray(m) * jnp.tile(v.reshape(F, 1), (T, 1))


def _prep_block(p, F):
    return {
        'u1': _padn(p['U1'].reshape(1, -1), 1),
        'u2': _padn(p['U2'], 1),
        'u3': _blockdiag(p['U3'], F),
        'be': p['be'][0],
        've': p['Ve'],
        'w1': p['W1'].reshape(T, 1),
        'w2': p['W2'],
        'w3': _blockdiag(p['W3'], F),
        'bs': _padn(_padn(p['bs'][0].T, 0), 1),
        'vs': _padn(_padn(p['Vs'].T, 0), 1),
        'cw': p['cw'].reshape(KCH * F, FC),
        'cb': p['cb'].reshape(1, FC),
        'tw': jnp.transpose(p['tw'][:, :, 0, :], (2, 1, 0)).reshape(3 * FC, FT),
        'tb': p['tb'].reshape(1, FT),
        'rw': p['rw'][:, :, 0, 0].T,
        'rb': p['rb'].reshape(1, FT),
        'lng': jnp.tile(p['lng'].reshape(1, FT), (1, T)),
        'lnb': jnp.tile(p['lnb'].reshape(1, FT), (1, T)),
    }


def _full_spec(shape):
    nz = len(shape)
    return pl.BlockSpec(shape, lambda b, _n=nz: (0,) * _n)


def kernel(x, edge_index, params):
    B = x.shape[0]
    E = edge_index.shape[1]
    EP = ((E + 127) // 128) * 128
    x3 = _padn(x[:, :, 0, :], 1)
    rowp = jnp.pad(edge_index[0], (0, EP - E)).reshape(1, EP)
    colp = jnp.pad(edge_index[1], (0, EP - E)).reshape(1, EP)

    A = _graph_adjacency_sc(rowp.reshape(EP), colp.reshape(EP), EP)

    b0 = _prep_block(params['b0'], 1)
    b1 = _prep_block(params['b1'], FT)
    fwr = jnp.transpose(params['fw'][:, :, 0, :], (1, 2, 0)).reshape(T * FT, P)
    fbr = params['fb'].reshape(1, P)
    import numpy as np
    negrow = jnp.asarray(
        np.where(np.arange(NP) < NREAL, 0.0, -1e9)[None, :].astype(np.float32))
    eye = jnp.asarray(np.eye(NP, dtype=np.float32))
    selc = np.arange(FT * T) // FT
    sel = jnp.asarray(
        (selc[:, None] == np.arange(T)[None, :]).astype(np.float32) / FT)
    bsel = jnp.asarray(
        (np.arange(T)[:, None] == selc[None, :]).astype(np.float32))
    flat = ([b0[k] for k in _BKEYS] + [b1[k] for k in _BKEYS] + [fwr, fbr]
            + [negrow, eye, sel, bsel])

    BPP = 2
    in_specs = [pl.BlockSpec((BPP, NP, T), lambda b: (b, 0, 0)),
                _full_spec((NP, NP))]
    in_specs += [_full_spec(a.shape) for a in flat]

    outp = pl.pallas_call(
        _main_kernel,
        grid=(B // BPP,),
        in_specs=in_specs,
        out_specs=pl.BlockSpec((BPP, NP, P), lambda b: (b, 0, 0)),
        out_shape=jax.ShapeDtypeStruct((B, NP, P), jnp.float32),
        compiler_params=pltpu.CompilerParams(
            dimension_semantics=("parallel",)),
    )(x3, A, *flat)
    return outp[:, :NREAL, :]
